# Initial kernel scaffold; baseline (speedup 1.0000x reference)
import jax
import jax.numpy as jnp
from jax.experimental import pallas as pl


def kernel(x, quadra_index, quadra_attr, W1, b1, W2, b2, W3, b3, Wout):
    raise NotImplementedError("write your pallas kernel here")


# speedup vs baseline: 2.9012x; 2.9012x over previous
import functools

import jax
import jax.numpy as jnp
from jax import lax
from jax.experimental import pallas as pl
from jax.experimental.pallas import tpu as pltpu
from jax.experimental.pallas import tpu_sc as plsc

N_NODES = 10000
N_DIH = 320000
SDIM = 128
HID = 64

NC = 2
NS = 16

G_EDGES_PT = N_DIH // NS
G_CH = 80
G_NCH = G_EDGES_PT // G_CH

_PREC = lax.Precision.HIGHEST


def _proj_body(x_ref, w1_ref, b1_ref, out_ref):
    xv = x_ref[...]
    a = jnp.dot(xv, w1_ref[0:SDIM, :], preferred_element_type=jnp.float32,
                precision=_PREC) + b1_ref[...]
    b = jnp.dot(xv, w1_ref[SDIM:2 * SDIM, :], preferred_element_type=jnp.float32,
                precision=_PREC)
    out_ref[:, :HID] = a
    out_ref[:, HID:] = b


_proj = pl.pallas_call(
    _proj_body,
    out_shape=jax.ShapeDtypeStruct((N_NODES, SDIM), jnp.float32),
)


def _sc_gather_body(tbl_hbm, gidx_hbm, out_hbm, idx_v, rows_v, sem):
    cid = lax.axis_index("c")
    sid = lax.axis_index("s")
    base = sid * G_EDGES_PT

    def chunk(i, carry):
        pltpu.sync_copy(gidx_hbm.at[cid, sid, i], idx_v)
        pltpu.async_copy(tbl_hbm.at[idx_v.at[0]], rows_v, sem).wait()
        pltpu.sync_copy(rows_v, out_hbm.at[cid, pl.ds(base + i * G_CH, G_CH)])
        return carry

    lax.fori_loop(0, G_NCH, chunk, 0)


@functools.cache
def _get_sc_gather():
    return pl.kernel(
        _sc_gather_body,
        out_type=jax.ShapeDtypeStruct((2, N_DIH, SDIM), jnp.float32),
        mesh=plsc.VectorSubcoreMesh(core_axis_name="c", subcore_axis_name="s",
                                    num_cores=NC, num_subcores=NS),
        scratch_types=[
            pltpu.VMEM((1, G_CH), jnp.int32),
            pltpu.VMEM((G_CH, SDIM), jnp.float32),
            pltpu.SemaphoreType.DMA,
        ],
    )


E_BLK = 4000


def _mlp_body(ga_ref, gb_ref, attr_ref, w1c_ref, w2_ref, b2_ref, out_ref):
    a = attr_ref[...]
    w1c = w1c_ref[...]
    t = ga_ref[0][:, :HID] + gb_ref[0][:, HID:]
    t = t + a[:, 0:1] * w1c[0:1, :]
    t = t + a[:, 1:2] * w1c[1:2, :]
    t = t + a[:, 2:3] * w1c[2:3, :]
    h1 = t * jax.nn.sigmoid(t)
    u = jnp.dot(h1, w2_ref[...], preferred_element_type=jnp.float32,
                precision=_PREC) + b2_ref[...]
    h2 = u * jax.nn.sigmoid(u)
    out_ref[:, :HID] = h2
    out_ref[:, HID:HID + 16] = jnp.ones((E_BLK, 16), jnp.float32)
    out_ref[:, HID + 16:] = jnp.zeros((E_BLK, SDIM - HID - 16), jnp.float32)


_mlp = pl.pallas_call(
    _mlp_body,
    grid=(N_DIH // E_BLK,),
    in_specs=[
        pl.BlockSpec((1, E_BLK, SDIM), lambda i: (0, i, 0)),
        pl.BlockSpec((1, E_BLK, SDIM), lambda i: (1, i, 0)),
        pl.BlockSpec((E_BLK, 3), lambda i: (i, 0)),
        pl.BlockSpec((3, HID), lambda i: (0, 0)),
        pl.BlockSpec((HID, HID), lambda i: (0, 0)),
        pl.BlockSpec((1, HID), lambda i: (0, 0)),
    ],
    out_specs=pl.BlockSpec((E_BLK, SDIM), lambda i: (i, 0)),
    out_shape=jax.ShapeDtypeStruct((N_DIH, SDIM), jnp.float32),
)


def _sc_scatter_body(h2_hbm, cidx_hbm, zs_hbm,
                     sout_hbm,
                     iv, rows_v, s_sh):
    cid = lax.axis_index("c")
    sid = lax.axis_index("s")

    @pl.when(sid == 0)
    def _():
        pltpu.sync_copy(zs_hbm, s_sh)

    plsc.subcore_barrier()
    base = sid * G_EDGES_PT

    def chunk(i, carry):
        pltpu.sync_copy(cidx_hbm.at[cid, sid, i], iv)
        pltpu.sync_copy(h2_hbm.at[pl.ds(base + i * G_CH, G_CH)], rows_v)
        pltpu.sync_copy(rows_v, s_sh.at[iv.at[0]], add=True)
        return carry

    lax.fori_loop(0, G_NCH, chunk, 0)
    plsc.subcore_barrier()

    @pl.when(sid == 0)
    def _():
        pltpu.sync_copy(s_sh, sout_hbm.at[cid])


@functools.cache
def _get_sc_scatter():
    return pl.kernel(
        _sc_scatter_body,
        out_type=jax.ShapeDtypeStruct((NC, N_NODES, SDIM), jnp.float32),
        mesh=plsc.VectorSubcoreMesh(core_axis_name="c", subcore_axis_name="s",
                                    num_cores=NC, num_subcores=NS),
        scratch_types=[
            pltpu.VMEM((1, G_CH), jnp.int32),
            pltpu.VMEM((G_CH, SDIM), jnp.float32),
            pltpu.VMEM_SHARED((N_NODES, SDIM), jnp.float32),
        ],
    )


def _final_body(s_ref, w3_ref, b3_ref, wout_ref, out_ref):
    sj = s_ref[0, :, :HID]
    sk = s_ref[1, :, :HID]
    cj = s_ref[0, :, HID:HID + 1]
    ck = s_ref[1, :, HID:HID + 1]
    y = sj / jnp.maximum(cj, 1.0) + sk / jnp.maximum(ck, 1.0)
    ind = (cj > 0.0).astype(jnp.float32) + (ck > 0.0).astype(jnp.float32)
    w3w = jnp.dot(w3_ref[...], wout_ref[...], preferred_element_type=jnp.float32,
                  precision=_PREC)
    b3w = jnp.dot(b3_ref[...], wout_ref[...], preferred_element_type=jnp.float32,
                  precision=_PREC)
    out = jnp.dot(y, w3w, preferred_element_type=jnp.float32,
                  precision=_PREC) + ind * b3w
    out_ref[...] = out * (0.5 / (SDIM ** 0.5))


_final = pl.pallas_call(
    _final_body,
    out_shape=jax.ShapeDtypeStruct((N_NODES, SDIM), jnp.float32),
)


def kernel(x, quadra_index, quadra_attr, W1, b1, W2, b2, W3, b3, Wout):
    f32 = jnp.float32
    b1r = b1.reshape(1, HID)
    b2r = b2.reshape(1, HID)
    b3r = b3.reshape(1, SDIM)

    tbl = _proj(x, W1, b1r)

    ii = quadra_index[0]
    jj = quadra_index[1]
    kk = quadra_index[2]
    ll = quadra_index[3]
    gidx = jnp.stack([ii, ll]).reshape(NC, NS, G_NCH, 1, G_CH)
    cidx = jnp.stack([jj, kk]).reshape(NC, NS, G_NCH, 1, G_CH)

    g = _get_sc_gather()(tbl, gidx)

    h2 = _mlp(g, g, quadra_attr, W1[2 * SDIM:], W2, b2r)

    zs = jnp.zeros((N_NODES, SDIM), f32)
    sacc = _get_sc_scatter()(h2, cidx, zs)

    return _final(sacc, W3, b3r, Wout)


# speedup vs baseline: 3.6310x; 1.2516x over previous
import functools

import jax
import jax.numpy as jnp
from jax import lax
from jax.experimental import pallas as pl
from jax.experimental.pallas import tpu as pltpu
from jax.experimental.pallas import tpu_sc as plsc

N_NODES = 10000
N_DIH = 320000
SDIM = 128
HID = 64

NC = 2
NS = 16

G_EDGES_PT = N_DIH // NS
G_CH = 80
G_NCH = G_EDGES_PT // G_CH
G_NRND = G_NCH // 2

_PREC = lax.Precision.HIGHEST


def _proj_body(x_ref, w1_ref, b1_ref, out_ref):
    xv = x_ref[...]
    a = jnp.dot(xv, w1_ref[0:SDIM, :], preferred_element_type=jnp.float32,
                precision=_PREC) + b1_ref[...]
    b = jnp.dot(xv, w1_ref[SDIM:2 * SDIM, :], preferred_element_type=jnp.float32,
                precision=_PREC)
    out_ref[:, :HID] = a
    out_ref[:, HID:] = b


_proj = pl.pallas_call(
    _proj_body,
    out_shape=jax.ShapeDtypeStruct((N_NODES, SDIM), jnp.float32),
)


def _sc_gather_body(tbl_hbm, gidx_hbm, out_hbm, idx_v, rows0, rows1,
                    sem_g, sem_w):
    cid = lax.axis_index("c")
    sid = lax.axis_index("s")
    base = sid * G_EDGES_PT

    def rnd(r, carry):
        @pl.when(r > 0)
        def _():
            pltpu.make_async_copy(rows0, out_hbm.at[cid, pl.ds(0, G_CH)],
                                  sem_w).wait()
            pltpu.make_async_copy(rows1, out_hbm.at[cid, pl.ds(0, G_CH)],
                                  sem_w).wait()

        pltpu.sync_copy(gidx_hbm.at[cid, sid, r], idx_v)
        c0 = pltpu.async_copy(tbl_hbm.at[idx_v.at[0]], rows0, sem_g)
        c1 = pltpu.async_copy(tbl_hbm.at[idx_v.at[1]], rows1, sem_g)
        c0.wait()
        c1.wait()
        o = base + 2 * r * G_CH
        pltpu.async_copy(rows0, out_hbm.at[cid, pl.ds(o, G_CH)], sem_w)
        pltpu.async_copy(rows1, out_hbm.at[cid, pl.ds(o + G_CH, G_CH)], sem_w)
        return carry

    lax.fori_loop(0, G_NRND, rnd, 0)
    pltpu.make_async_copy(rows0, out_hbm.at[cid, pl.ds(0, G_CH)], sem_w).wait()
    pltpu.make_async_copy(rows1, out_hbm.at[cid, pl.ds(0, G_CH)], sem_w).wait()


@functools.cache
def _get_sc_gather():
    return pl.kernel(
        _sc_gather_body,
        out_type=jax.ShapeDtypeStruct((2, N_DIH, SDIM), jnp.float32),
        mesh=plsc.VectorSubcoreMesh(core_axis_name="c", subcore_axis_name="s",
                                    num_cores=NC, num_subcores=NS),
        scratch_types=[
            pltpu.VMEM((2, G_CH), jnp.int32),
            pltpu.VMEM((G_CH, SDIM), jnp.float32),
            pltpu.VMEM((G_CH, SDIM), jnp.float32),
            pltpu.SemaphoreType.DMA,
            pltpu.SemaphoreType.DMA,
        ],
    )


E_BLK = 4000


def _mlp_body(ga_ref, gb_ref, attr_ref, w1c_ref, w2_ref, b2_ref, out_ref):
    a = attr_ref[...]
    w1c = w1c_ref[...]
    t = ga_ref[0][:, :HID] + gb_ref[0][:, HID:]
    t = t + a[:, 0:1] * w1c[0:1, :]
    t = t + a[:, 1:2] * w1c[1:2, :]
    t = t + a[:, 2:3] * w1c[2:3, :]
    h1 = t * jax.nn.sigmoid(t)
    u = jnp.dot(h1, w2_ref[...], preferred_element_type=jnp.float32,
                precision=_PREC) + b2_ref[...]
    h2 = u * jax.nn.sigmoid(u)
    out_ref[:, :HID] = h2
    out_ref[:, HID:HID + 16] = jnp.ones((E_BLK, 16), jnp.float32)
    out_ref[:, HID + 16:] = jnp.zeros((E_BLK, SDIM - HID - 16), jnp.float32)


_mlp = pl.pallas_call(
    _mlp_body,
    grid=(N_DIH // E_BLK,),
    in_specs=[
        pl.BlockSpec((1, E_BLK, SDIM), lambda i: (0, i, 0)),
        pl.BlockSpec((1, E_BLK, SDIM), lambda i: (1, i, 0)),
        pl.BlockSpec((E_BLK, 3), lambda i: (i, 0)),
        pl.BlockSpec((3, HID), lambda i: (0, 0)),
        pl.BlockSpec((HID, HID), lambda i: (0, 0)),
        pl.BlockSpec((1, HID), lambda i: (0, 0)),
    ],
    out_specs=pl.BlockSpec((E_BLK, SDIM), lambda i: (i, 0)),
    out_shape=jax.ShapeDtypeStruct((N_DIH, SDIM), jnp.float32),
)


def _sc_scatter_body(h2_hbm, cidx_hbm, zs_hbm,
                     sout_hbm,
                     iv, rows0, rows1, s_sh, sem_l, sem_s):
    cid = lax.axis_index("c")
    sid = lax.axis_index("s")

    @pl.when(sid == 0)
    def _():
        pltpu.sync_copy(zs_hbm, s_sh)

    plsc.subcore_barrier()
    base = sid * G_EDGES_PT

    def rnd(r, carry):
        @pl.when(r > 0)
        def _():
            pltpu.make_async_copy(zs_hbm.at[pl.ds(0, G_CH)], rows0,
                                  sem_s).wait()
            pltpu.make_async_copy(zs_hbm.at[pl.ds(0, G_CH)], rows1,
                                  sem_s).wait()

        pltpu.sync_copy(cidx_hbm.at[cid, sid, r], iv)
        o = base + 2 * r * G_CH
        l0 = pltpu.async_copy(h2_hbm.at[pl.ds(o, G_CH)], rows0, sem_l)
        l1 = pltpu.async_copy(h2_hbm.at[pl.ds(o + G_CH, G_CH)], rows1, sem_l)
        l0.wait()
        l1.wait()
        pltpu.async_copy(rows0, s_sh.at[iv.at[0]], sem_s, add=True)
        pltpu.async_copy(rows1, s_sh.at[iv.at[1]], sem_s, add=True)
        return carry

    lax.fori_loop(0, G_NRND, rnd, 0)
    pltpu.make_async_copy(zs_hbm.at[pl.ds(0, G_CH)], rows0, sem_s).wait()
    pltpu.make_async_copy(zs_hbm.at[pl.ds(0, G_CH)], rows1, sem_s).wait()
    plsc.subcore_barrier()

    @pl.when(sid == 0)
    def _():
        pltpu.sync_copy(s_sh, sout_hbm.at[cid])


@functools.cache
def _get_sc_scatter():
    return pl.kernel(
        _sc_scatter_body,
        out_type=jax.ShapeDtypeStruct((NC, N_NODES, SDIM), jnp.float32),
        mesh=plsc.VectorSubcoreMesh(core_axis_name="c", subcore_axis_name="s",
                                    num_cores=NC, num_subcores=NS),
        scratch_types=[
            pltpu.VMEM((2, G_CH), jnp.int32),
            pltpu.VMEM((G_CH, SDIM), jnp.float32),
            pltpu.VMEM((G_CH, SDIM), jnp.float32),
            pltpu.VMEM_SHARED((N_NODES, SDIM), jnp.float32),
            pltpu.SemaphoreType.DMA,
            pltpu.SemaphoreType.DMA,
        ],
    )


def _final_body(s_ref, w3_ref, b3_ref, wout_ref, out_ref):
    sj = s_ref[0, :, :HID]
    sk = s_ref[1, :, :HID]
    cj = s_ref[0, :, HID:HID + 1]
    ck = s_ref[1, :, HID:HID + 1]
    y = sj / jnp.maximum(cj, 1.0) + sk / jnp.maximum(ck, 1.0)
    ind = (cj > 0.0).astype(jnp.float32) + (ck > 0.0).astype(jnp.float32)
    w3w = jnp.dot(w3_ref[...], wout_ref[...], preferred_element_type=jnp.float32,
                  precision=_PREC)
    b3w = jnp.dot(b3_ref[...], wout_ref[...], preferred_element_type=jnp.float32,
                  precision=_PREC)
    out = jnp.dot(y, w3w, preferred_element_type=jnp.float32,
                  precision=_PREC) + ind * b3w
    out_ref[...] = out * (0.5 / (SDIM ** 0.5))


_final = pl.pallas_call(
    _final_body,
    out_shape=jax.ShapeDtypeStruct((N_NODES, SDIM), jnp.float32),
)


def kernel(x, quadra_index, quadra_attr, W1, b1, W2, b2, W3, b3, Wout):
    f32 = jnp.float32
    b1r = b1.reshape(1, HID)
    b2r = b2.reshape(1, HID)
    b3r = b3.reshape(1, SDIM)

    tbl = _proj(x, W1, b1r)

    ii = quadra_index[0]
    jj = quadra_index[1]
    kk = quadra_index[2]
    ll = quadra_index[3]
    gidx = jnp.stack([ii, ll]).reshape(NC, NS, G_NRND, 2, G_CH)
    cidx = jnp.stack([jj, kk]).reshape(NC, NS, G_NRND, 2, G_CH)

    g = _get_sc_gather()(tbl, gidx)

    h2 = _mlp(g, g, quadra_attr, W1[2 * SDIM:], W2, b2r)

    zs = jnp.zeros((N_NODES, SDIM), f32)
    sacc = _get_sc_scatter()(h2, cidx, zs)

    return _final(sacc, W3, b3r, Wout)


# speedup vs baseline: 4.0091x; 1.1041x over previous
import functools

import jax
import jax.numpy as jnp
from jax import lax
from jax.experimental import pallas as pl
from jax.experimental.pallas import tpu as pltpu
from jax.experimental.pallas import tpu_sc as plsc

N_NODES = 10000
N_DIH = 320000
SDIM = 128
HID = 64

NC = 2
NS = 16

G_EDGES_PT = N_DIH // NS
G_CH = 80
G_NCH = G_EDGES_PT // G_CH
G_SLAB = 10
G_NSLAB = G_NCH // G_SLAB
G_SLAB_ROWS = G_SLAB * G_CH
S_SLAB = 2
S_NSLAB = G_NCH // S_SLAB
S_SLAB_ROWS = S_SLAB * G_CH

_PREC = lax.Precision.HIGHEST


def _proj_body(x_ref, w1_ref, b1_ref, out_ref):
    xv = x_ref[...]
    a = jnp.dot(xv, w1_ref[0:SDIM, :], preferred_element_type=jnp.float32,
                precision=_PREC) + b1_ref[...]
    b = jnp.dot(xv, w1_ref[SDIM:2 * SDIM, :], preferred_element_type=jnp.float32,
                precision=_PREC)
    out_ref[:, :HID] = a
    out_ref[:, HID:] = b


_proj = pl.pallas_call(
    _proj_body,
    out_shape=jax.ShapeDtypeStruct((N_NODES, SDIM), jnp.float32),
)


def _sc_gather_body(tbl_hbm, gidx_hbm, out_hbm, idx_v, rows_v, sem_g, sem_w):
    cid = lax.axis_index("c")
    sid = lax.axis_index("s")
    base = sid * G_EDGES_PT

    def slab(p, carry):
        @pl.when(p > 0)
        def _():
            pltpu.make_async_copy(rows_v, out_hbm.at[cid, pl.ds(0, G_SLAB_ROWS)],
                                  sem_w).wait()

        pltpu.sync_copy(gidx_hbm.at[cid, sid, p], idx_v)
        for u in range(G_SLAB):
            pltpu.async_copy(tbl_hbm.at[idx_v.at[u]],
                             rows_v.at[pl.ds(u * G_CH, G_CH)], sem_g)
        for u in range(G_SLAB):
            pltpu.make_async_copy(tbl_hbm.at[idx_v.at[u]],
                                  rows_v.at[pl.ds(u * G_CH, G_CH)],
                                  sem_g).wait()
        pltpu.async_copy(rows_v,
                         out_hbm.at[cid, pl.ds(base + p * G_SLAB_ROWS,
                                               G_SLAB_ROWS)], sem_w)
        return carry

    lax.fori_loop(0, G_NSLAB, slab, 0)
    pltpu.make_async_copy(rows_v, out_hbm.at[cid, pl.ds(0, G_SLAB_ROWS)],
                          sem_w).wait()


@functools.cache
def _get_sc_gather():
    return pl.kernel(
        _sc_gather_body,
        out_type=jax.ShapeDtypeStruct((2, N_DIH, SDIM), jnp.float32),
        mesh=plsc.VectorSubcoreMesh(core_axis_name="c", subcore_axis_name="s",
                                    num_cores=NC, num_subcores=NS),
        scratch_types=[
            pltpu.VMEM((G_SLAB, G_CH), jnp.int32),
            pltpu.VMEM((G_SLAB_ROWS, SDIM), jnp.float32),
            pltpu.SemaphoreType.DMA,
            pltpu.SemaphoreType.DMA,
        ],
    )


E_BLK = 4000


def _mlp_body(ga_ref, gb_ref, attr_ref, w1c_ref, w2_ref, b2_ref, out_ref):
    a = attr_ref[...]
    w1c = w1c_ref[...]
    t = ga_ref[0][:, :HID] + gb_ref[0][:, HID:]
    t = t + a[:, 0:1] * w1c[0:1, :]
    t = t + a[:, 1:2] * w1c[1:2, :]
    t = t + a[:, 2:3] * w1c[2:3, :]
    h1 = t * jax.nn.sigmoid(t)
    u = jnp.dot(h1, w2_ref[...], preferred_element_type=jnp.float32,
                precision=_PREC) + b2_ref[...]
    h2 = u * jax.nn.sigmoid(u)
    out_ref[:, :HID] = h2
    out_ref[:, HID:HID + 16] = jnp.ones((E_BLK, 16), jnp.float32)
    out_ref[:, HID + 16:] = jnp.zeros((E_BLK, SDIM - HID - 16), jnp.float32)


_mlp = pl.pallas_call(
    _mlp_body,
    grid=(N_DIH // E_BLK,),
    in_specs=[
        pl.BlockSpec((1, E_BLK, SDIM), lambda i: (0, i, 0)),
        pl.BlockSpec((1, E_BLK, SDIM), lambda i: (1, i, 0)),
        pl.BlockSpec((E_BLK, 3), lambda i: (i, 0)),
        pl.BlockSpec((3, HID), lambda i: (0, 0)),
        pl.BlockSpec((HID, HID), lambda i: (0, 0)),
        pl.BlockSpec((1, HID), lambda i: (0, 0)),
    ],
    out_specs=pl.BlockSpec((E_BLK, SDIM), lambda i: (i, 0)),
    out_shape=jax.ShapeDtypeStruct((N_DIH, SDIM), jnp.float32),
)


def _sc_scatter_body(h2_hbm, cidx_hbm, zs_hbm,
                     sout_hbm,
                     iv, rows_v, s_sh, sem_l, sem_s):
    cid = lax.axis_index("c")
    sid = lax.axis_index("s")

    @pl.when(sid == 0)
    def _():
        pltpu.sync_copy(zs_hbm, s_sh)

    plsc.subcore_barrier()
    base = sid * G_EDGES_PT

    def slab(p, carry):
        @pl.when(p > 0)
        def _():
            for _u in range(S_SLAB):
                pltpu.make_async_copy(zs_hbm.at[pl.ds(0, G_CH)],
                                      rows_v.at[pl.ds(0, G_CH)], sem_s).wait()

        pltpu.sync_copy(cidx_hbm.at[cid, sid, p], iv)
        pltpu.async_copy(h2_hbm.at[pl.ds(base + p * S_SLAB_ROWS, S_SLAB_ROWS)],
                         rows_v, sem_l).wait()
        for u in range(S_SLAB):
            pltpu.async_copy(rows_v.at[pl.ds(u * G_CH, G_CH)],
                             s_sh.at[iv.at[u]], sem_s, add=True)
        return carry

    lax.fori_loop(0, S_NSLAB, slab, 0)
    for _u in range(S_SLAB):
        pltpu.make_async_copy(zs_hbm.at[pl.ds(0, G_CH)],
                              rows_v.at[pl.ds(0, G_CH)], sem_s).wait()
    plsc.subcore_barrier()

    @pl.when(sid == 0)
    def _():
        pltpu.sync_copy(s_sh, sout_hbm.at[cid])


@functools.cache
def _get_sc_scatter():
    return pl.kernel(
        _sc_scatter_body,
        out_type=jax.ShapeDtypeStruct((NC, N_NODES, SDIM), jnp.float32),
        mesh=plsc.VectorSubcoreMesh(core_axis_name="c", subcore_axis_name="s",
                                    num_cores=NC, num_subcores=NS),
        scratch_types=[
            pltpu.VMEM((S_SLAB, G_CH), jnp.int32),
            pltpu.VMEM((S_SLAB_ROWS, SDIM), jnp.float32),
            pltpu.VMEM_SHARED((N_NODES, SDIM), jnp.float32),
            pltpu.SemaphoreType.DMA,
            pltpu.SemaphoreType.DMA,
        ],
    )


def _final_body(s_ref, w3_ref, b3_ref, wout_ref, out_ref):
    sj = s_ref[0, :, :HID]
    sk = s_ref[1, :, :HID]
    cj = s_ref[0, :, HID:HID + 1]
    ck = s_ref[1, :, HID:HID + 1]
    y = sj / jnp.maximum(cj, 1.0) + sk / jnp.maximum(ck, 1.0)
    ind = (cj > 0.0).astype(jnp.float32) + (ck > 0.0).astype(jnp.float32)
    w3w = jnp.dot(w3_ref[...], wout_ref[...], preferred_element_type=jnp.float32,
                  precision=_PREC)
    b3w = jnp.dot(b3_ref[...], wout_ref[...], preferred_element_type=jnp.float32,
                  precision=_PREC)
    out = jnp.dot(y, w3w, preferred_element_type=jnp.float32,
                  precision=_PREC) + ind * b3w
    out_ref[...] = out * (0.5 / (SDIM ** 0.5))


_final = pl.pallas_call(
    _final_body,
    out_shape=jax.ShapeDtypeStruct((N_NODES, SDIM), jnp.float32),
)


def kernel(x, quadra_index, quadra_attr, W1, b1, W2, b2, W3, b3, Wout):
    f32 = jnp.float32
    b1r = b1.reshape(1, HID)
    b2r = b2.reshape(1, HID)
    b3r = b3.reshape(1, SDIM)

    tbl = _proj(x, W1, b1r)

    ii = quadra_index[0]
    jj = quadra_index[1]
    kk = quadra_index[2]
    ll = quadra_index[3]
    gidx = jnp.stack([ii, ll]).reshape(NC, NS, G_NSLAB, G_SLAB, G_CH)
    cidx = jnp.stack([jj, kk]).reshape(NC, NS, S_NSLAB, S_SLAB, G_CH)

    g = _get_sc_gather()(tbl, gidx)

    h2 = _mlp(g, g, quadra_attr, W1[2 * SDIM:], W2, b2r)

    zs = jnp.zeros((N_NODES, SDIM), f32)
    sacc = _get_sc_scatter()(h2, cidx, zs)

    return _final(sacc, W3, b3r, Wout)


# speedup vs baseline: 4.3832x; 1.0933x over previous
import functools

import jax
import jax.numpy as jnp
from jax import lax
from jax.experimental import pallas as pl
from jax.experimental.pallas import tpu as pltpu
from jax.experimental.pallas import tpu_sc as plsc

N_NODES = 10000
N_DIH = 320000
SDIM = 128
HID = 64

NC = 2
NS = 16

G_EDGES_PT = N_DIH // NS
G_CH = 80
G_NCH = G_EDGES_PT // G_CH
G_SLAB = 10
G_NSLAB = G_NCH // G_SLAB
G_SLAB_ROWS = G_SLAB * G_CH
S_SLAB = 2
S_NSLAB = G_NCH // S_SLAB
S_SLAB_ROWS = S_SLAB * G_CH

_PREC = lax.Precision.HIGHEST


def _proj_body(x_ref, w1_ref, b1_ref, out_ref):
    xv = x_ref[...]
    a = jnp.dot(xv, w1_ref[0:SDIM, :], preferred_element_type=jnp.float32,
                precision=_PREC) + b1_ref[...]
    b = jnp.dot(xv, w1_ref[SDIM:2 * SDIM, :], preferred_element_type=jnp.float32,
                precision=_PREC)
    out_ref[:, :HID] = a
    out_ref[:, HID:] = b


_proj = pl.pallas_call(
    _proj_body,
    out_shape=jax.ShapeDtypeStruct((N_NODES, SDIM), jnp.float32),
)


def _sc_gather_body(tbl_hbm, gidx_hbm, out_hbm, idx_v, rows_v, sem_g, sem_w):
    cid = lax.axis_index("c")
    sid = lax.axis_index("s")
    base = sid * G_EDGES_PT

    def slab(p, carry):
        @pl.when(p > 0)
        def _():
            pltpu.make_async_copy(rows_v, out_hbm.at[cid, pl.ds(0, G_SLAB_ROWS)],
                                  sem_w).wait()

        pltpu.sync_copy(gidx_hbm.at[cid, sid, p], idx_v)
        for u in range(G_SLAB):
            pltpu.async_copy(tbl_hbm.at[idx_v.at[u]],
                             rows_v.at[pl.ds(u * G_CH, G_CH)], sem_g)
        for u in range(G_SLAB):
            pltpu.make_async_copy(tbl_hbm.at[idx_v.at[u]],
                                  rows_v.at[pl.ds(u * G_CH, G_CH)],
                                  sem_g).wait()
        pltpu.async_copy(rows_v,
                         out_hbm.at[cid, pl.ds(base + p * G_SLAB_ROWS,
                                               G_SLAB_ROWS)], sem_w)
        return carry

    lax.fori_loop(0, G_NSLAB, slab, 0)
    pltpu.make_async_copy(rows_v, out_hbm.at[cid, pl.ds(0, G_SLAB_ROWS)],
                          sem_w).wait()


@functools.cache
def _get_sc_gather():
    return pl.kernel(
        _sc_gather_body,
        out_type=jax.ShapeDtypeStruct((2, N_DIH, SDIM), jnp.float32),
        mesh=plsc.VectorSubcoreMesh(core_axis_name="c", subcore_axis_name="s",
                                    num_cores=NC, num_subcores=NS),
        scratch_types=[
            pltpu.VMEM((G_SLAB, G_CH), jnp.int32),
            pltpu.VMEM((G_SLAB_ROWS, SDIM), jnp.float32),
            pltpu.SemaphoreType.DMA,
            pltpu.SemaphoreType.DMA,
        ],
    )


E_BLK = 4000


def _mlp_body(ga_ref, gb_ref, attr_ref, w1c_ref, w2_ref, b2_ref, out_ref):
    a = attr_ref[...]
    w1c = w1c_ref[...]
    t = ga_ref[0][:, :HID] + gb_ref[0][:, HID:]
    t = t + a[:, 0:1] * w1c[0:1, :]
    t = t + a[:, 1:2] * w1c[1:2, :]
    t = t + a[:, 2:3] * w1c[2:3, :]
    h1 = t * jax.nn.sigmoid(t)
    u = jnp.dot(h1, w2_ref[...], preferred_element_type=jnp.float32,
                precision=_PREC) + b2_ref[...]
    h2 = u * jax.nn.sigmoid(u)
    out_ref[:, :HID] = h2
    out_ref[:, HID:HID + 16] = jnp.ones((E_BLK, 16), jnp.float32)
    out_ref[:, HID + 16:] = jnp.zeros((E_BLK, SDIM - HID - 16), jnp.float32)


_mlp = pl.pallas_call(
    _mlp_body,
    grid=(N_DIH // E_BLK,),
    in_specs=[
        pl.BlockSpec((1, E_BLK, SDIM), lambda i: (0, i, 0)),
        pl.BlockSpec((1, E_BLK, SDIM), lambda i: (1, i, 0)),
        pl.BlockSpec((E_BLK, 3), lambda i: (i, 0)),
        pl.BlockSpec((3, HID), lambda i: (0, 0)),
        pl.BlockSpec((HID, HID), lambda i: (0, 0)),
        pl.BlockSpec((1, HID), lambda i: (0, 0)),
    ],
    out_specs=pl.BlockSpec((E_BLK, SDIM), lambda i: (i, 0)),
    out_shape=jax.ShapeDtypeStruct((N_DIH, SDIM), jnp.float32),
)


def _sc_scatter_body(h2_hbm, cidx_hbm, zs_hbm,
                     sout_hbm,
                     iv_a, iv_b, rows_a, rows_b, s_sh,
                     sem_l, sem_sa, sem_sb):
    cid = lax.axis_index("c")
    sid = lax.axis_index("s")

    @pl.when(sid == 0)
    def _():
        pltpu.sync_copy(zs_hbm, s_sh)

    plsc.subcore_barrier()
    base = sid * G_EDGES_PT

    def _drain(sem):
        for _u in range(S_SLAB):
            pltpu.make_async_copy(zs_hbm.at[pl.ds(0, G_CH)],
                                  rows_a.at[pl.ds(0, G_CH)], sem).wait()

    def _do_slab(s, iv, rows_v, sem_s):
        pltpu.sync_copy(cidx_hbm.at[cid, sid, s], iv)
        pltpu.async_copy(h2_hbm.at[pl.ds(base + s * S_SLAB_ROWS, S_SLAB_ROWS)],
                         rows_v, sem_l).wait()
        for u in range(S_SLAB):
            pltpu.async_copy(rows_v.at[pl.ds(u * G_CH, G_CH)],
                             s_sh.at[iv.at[u]], sem_s, add=True)

    def pair(t, carry):
        @pl.when(t > 0)
        def _():
            _drain(sem_sa)

        _do_slab(2 * t, iv_a, rows_a, sem_sa)

        @pl.when(t > 0)
        def _():
            _drain(sem_sb)

        _do_slab(2 * t + 1, iv_b, rows_b, sem_sb)
        return carry

    lax.fori_loop(0, S_NSLAB // 2, pair, 0)
    _drain(sem_sa)
    _do_slab(S_NSLAB - 1, iv_a, rows_a, sem_sa)
    _drain(sem_sa)
    _drain(sem_sb)
    plsc.subcore_barrier()

    @pl.when(sid == 0)
    def _():
        pltpu.sync_copy(s_sh, sout_hbm.at[cid])


@functools.cache
def _get_sc_scatter():
    return pl.kernel(
        _sc_scatter_body,
        out_type=jax.ShapeDtypeStruct((NC, N_NODES, SDIM), jnp.float32),
        mesh=plsc.VectorSubcoreMesh(core_axis_name="c", subcore_axis_name="s",
                                    num_cores=NC, num_subcores=NS),
        scratch_types=[
            pltpu.VMEM((S_SLAB, G_CH), jnp.int32),
            pltpu.VMEM((S_SLAB, G_CH), jnp.int32),
            pltpu.VMEM((S_SLAB_ROWS, SDIM), jnp.float32),
            pltpu.VMEM((S_SLAB_ROWS, SDIM), jnp.float32),
            pltpu.VMEM_SHARED((N_NODES, SDIM), jnp.float32),
            pltpu.SemaphoreType.DMA,
            pltpu.SemaphoreType.DMA,
            pltpu.SemaphoreType.DMA,
        ],
    )


def _final_body(s_ref, w3_ref, b3_ref, wout_ref, out_ref):
    sj = s_ref[0, :, :HID]
    sk = s_ref[1, :, :HID]
    cj = s_ref[0, :, HID:HID + 1]
    ck = s_ref[1, :, HID:HID + 1]
    y = sj / jnp.maximum(cj, 1.0) + sk / jnp.maximum(ck, 1.0)
    ind = (cj > 0.0).astype(jnp.float32) + (ck > 0.0).astype(jnp.float32)
    w3w = jnp.dot(w3_ref[...], wout_ref[...], preferred_element_type=jnp.float32,
                  precision=_PREC)
    b3w = jnp.dot(b3_ref[...], wout_ref[...], preferred_element_type=jnp.float32,
                  precision=_PREC)
    out = jnp.dot(y, w3w, preferred_element_type=jnp.float32,
                  precision=_PREC) + ind * b3w
    out_ref[...] = out * (0.5 / (SDIM ** 0.5))


_final = pl.pallas_call(
    _final_body,
    out_shape=jax.ShapeDtypeStruct((N_NODES, SDIM), jnp.float32),
)


def kernel(x, quadra_index, quadra_attr, W1, b1, W2, b2, W3, b3, Wout):
    f32 = jnp.float32
    b1r = b1.reshape(1, HID)
    b2r = b2.reshape(1, HID)
    b3r = b3.reshape(1, SDIM)

    tbl = _proj(x, W1, b1r)

    ii = quadra_index[0]
    jj = quadra_index[1]
    kk = quadra_index[2]
    ll = quadra_index[3]
    gidx = jnp.stack([ii, ll]).reshape(NC, NS, G_NSLAB, G_SLAB, G_CH)
    cidx = jnp.stack([jj, kk]).reshape(NC, NS, S_NSLAB, S_SLAB, G_CH)

    g = _get_sc_gather()(tbl, gidx)

    h2 = _mlp(g, g, quadra_attr, W1[2 * SDIM:], W2, b2r)

    zs = jnp.zeros((N_NODES, SDIM), f32)
    sacc = _get_sc_scatter()(h2, cidx, zs)

    return _final(sacc, W3, b3r, Wout)
